# Initial kernel scaffold; baseline (speedup 1.0000x reference)
import jax
import jax.numpy as jnp
from jax.experimental import pallas as pl


def kernel(x_feature, adjacency_matrix, weight, bias):
    raise NotImplementedError("write your pallas kernel here")


# speedup vs baseline: 1.0392x; 1.0392x over previous
import jax
import jax.numpy as jnp
from jax.experimental import pallas as pl
from jax.experimental.pallas import tpu as pltpu

_BM = 400


def _gcn_body(x_ref, w_ref, a_ref, b_ref, o_ref, s_ref):
    @pl.when(pl.program_id(0) == 0)
    def _():
        s_ref[...] = jnp.dot(
            x_ref[...], w_ref[...], preferred_element_type=jnp.float32
        ).astype(jnp.bfloat16)

    acc = jnp.dot(
        a_ref[...].astype(jnp.bfloat16),
        s_ref[...],
        preferred_element_type=jnp.float32,
    )
    o_ref[...] = acc + b_ref[...]


def kernel(x_feature, adjacency_matrix, weight, bias):
    n, in_dim = x_feature.shape
    out_dim = weight.shape[1]
    bias2 = bias.reshape(1, out_dim)
    return pl.pallas_call(
        _gcn_body,
        grid=(n // _BM,),
        in_specs=[
            pl.BlockSpec((n, in_dim), lambda i: (0, 0)),
            pl.BlockSpec((in_dim, out_dim), lambda i: (0, 0)),
            pl.BlockSpec((_BM, n), lambda i: (i, 0)),
            pl.BlockSpec((1, out_dim), lambda i: (0, 0)),
        ],
        out_specs=pl.BlockSpec((_BM, out_dim), lambda i: (i, 0)),
        out_shape=jax.ShapeDtypeStruct((n, out_dim), jnp.float32),
        scratch_shapes=[pltpu.VMEM((n, out_dim), jnp.bfloat16)],
        compiler_params=pltpu.CompilerParams(
            dimension_semantics=("arbitrary",),
        ),
    )(x_feature, weight, adjacency_matrix, bias2)
